# baseline (speedup 1.0000x reference)
import functools

import jax
import jax.numpy as jnp
from jax import lax
from jax.experimental import pallas as pl
from jax.experimental.pallas import tpu as pltpu
from jax.experimental.pallas import tpu_sc as plsc

_B = 16384
_HIST = 50
_D = 64
_NC = 2
_NS = 16
_NW = _NC * _NS
_BAGS_PER_W = _B // _NW
_QW = 128
_NQ = _BAGS_PER_W // _QW
_NCHUNK = _HIST * _NQ
_NREG = _D // 16


def _sc_embedding_sum(xt, table, emb_bias):
    mesh = plsc.VectorSubcoreMesh(
        core_axis_name="c", subcore_axis_name="s",
        num_cores=_NC, num_subcores=_NS,
    )

    @functools.partial(
        pl.kernel,
        out_type=jax.ShapeDtypeStruct((_B, _D), jnp.float32),
        mesh=mesh,
        compiler_params=pltpu.CompilerParams(use_tc_tiling_on_sc=False),
        scratch_types=[
            pltpu.VMEM((_HIST, _BAGS_PER_W), jnp.int32),
            pltpu.VMEM((_QW, _D), jnp.float32),
            pltpu.VMEM((_QW, _D), jnp.float32),
            pltpu.VMEM((_BAGS_PER_W, _D), jnp.float32),
            pltpu.VMEM((_D,), jnp.float32),
            pltpu.SemaphoreType.DMA,
            pltpu.SemaphoreType.DMA,
        ],
    )
    def k(xt_hbm, tab_hbm, bias_hbm, out_hbm,
          idx_v, rows0, rows1, out_v, bias_v, sem0, sem1):
        wid = lax.axis_index("s") * _NC + lax.axis_index("c")
        base = wid * _BAGS_PER_W
        pltpu.sync_copy(xt_hbm.at[:, pl.ds(base, _BAGS_PER_W)], idx_v)
        pltpu.sync_copy(bias_hbm, bias_v)
        bias_regs = [bias_v[pl.ds(16 * g, 16)] for g in range(_NREG)]

        def init_body(b, carry):
            for g in range(_NREG):
                out_v[b, pl.ds(16 * g, 16)] = bias_regs[g]
            return carry

        lax.fori_loop(0, _BAGS_PER_W, init_body, 0)

        def idx_slice(cid):
            l = cid // _NQ
            qb = (cid % _NQ) * _QW
            return idx_v.at[l, pl.ds(qb, _QW)]

        def start(cid, rows, sem):
            pltpu.async_copy(tab_hbm.at[idx_slice(cid)], rows, sem)

        def wait(cid, rows, sem):
            pltpu.make_async_copy(tab_hbm.at[idx_slice(cid)], rows, sem).wait()

        def accumulate(cid, rows):
            qb = (cid % _NQ) * _QW
            for r in range(_QW):
                for g in range(_NREG):
                    plsc.addupdate(out_v.at[qb + r, pl.ds(16 * g, 16)],
                                   rows[r, pl.ds(16 * g, 16)])

        start(0, rows0, sem0)

        def step(i, carry):
            cid = 2 * i
            start(cid + 1, rows1, sem1)
            wait(cid, rows0, sem0)
            accumulate(cid, rows0)

            @pl.when(cid + 2 < _NCHUNK)
            def _prefetch():
                start(cid + 2, rows0, sem0)

            wait(cid + 1, rows1, sem1)
            accumulate(cid + 1, rows1)
            return carry

        lax.fori_loop(0, _NCHUNK // 2, step, 0)
        pltpu.sync_copy(out_v, out_hbm.at[pl.ds(base, _BAGS_PER_W)])

    return k(xt, table, emb_bias)


def kernel(x, table, emb_bias):
    xt = x.astype(jnp.int32).T
    return _sc_embedding_sum(xt, table, emb_bias)
